# Initial kernel scaffold; baseline (speedup 1.0000x reference)
import jax
import jax.numpy as jnp
from jax.experimental import pallas as pl


def kernel(x, edge_index, W1, b1, W2, b2):
    raise NotImplementedError("write your pallas kernel here")


# speedup vs baseline: 13.8534x; 13.8534x over previous
import functools
import jax
import jax.numpy as jnp
from jax import lax
from jax.experimental import pallas as pl
from jax.experimental.pallas import tpu as pltpu
from jax.experimental.pallas import tpu_sc as plsc

N = 10000
E = 320000
F = 128
FH = F // 2
NC = 2
NS = 16
NW = NC * NS
EPW = E // NW
CHUNK = 80
NCHUNK = EPW // CHUNK
DR = 624
ZR = 208
TAIL = N - NS * DR
DEGW = 16

ROWS_BLK = 1000
TC_GRID = N // ROWS_BLK


def _zero_vmem(ref, nrows, width):
  per_row = width // 16

  def body(i, _):
    r = i // per_row
    k = i % per_row
    ref[r, pl.ds(k * 16, 16)] = jnp.zeros((16,), jnp.float32)
    return 0

  lax.fori_loop(0, nrows * per_row, body, 0)


def _fill_ones(ref, nrows):
  def body(i, _):
    ref[i, :] = jnp.ones((16,), jnp.float32)
    return 0

  lax.fori_loop(0, nrows, body, 0)


def _zero_shared(zero_v, acc_sh, s):
  def body(i, _):
    pltpu.sync_copy(zero_v, acc_sh.at[pl.ds(s * DR + i * ZR, ZR)])
    return 0

  lax.fori_loop(0, DR // ZR, body, 0)

  @pl.when(s == NS - 1)
  def _():
    pltpu.sync_copy(zero_v.at[pl.ds(0, TAIL)], acc_sh.at[pl.ds(NS * DR, TAIL)])


def _drain_shared(acc_sh, out_hbm, c, s):
  pltpu.sync_copy(
      acc_sh.at[pl.ds(s * DR, DR)], out_hbm.at[pl.ds(c * N + s * DR, DR)])

  @pl.when(s == NS - 1)
  def _():
    pltpu.sync_copy(
        acc_sh.at[pl.ds(NS * DR, TAIL)],
        out_hbm.at[pl.ds(c * N + NS * DR, TAIL)])


def _mesh():
  return plsc.VectorSubcoreMesh(
      core_axis_name="c", subcore_axis_name="s", num_cores=NC,
      num_subcores=NS)


def _sc_degree(dst3):
  @functools.partial(
      pl.kernel,
      out_type=jax.ShapeDtypeStruct((NC * N, DEGW), jnp.float32),
      mesh=_mesh(),
      compiler_params=pltpu.CompilerParams(use_tc_tiling_on_sc=False),
      scratch_types=[
          pltpu.VMEM((NCHUNK, CHUNK), jnp.int32),
          pltpu.VMEM((CHUNK, DEGW), jnp.float32),
          pltpu.VMEM((ZR, DEGW), jnp.float32),
          pltpu.VMEM_SHARED((N, DEGW), jnp.float32),
      ],
  )
  def deg_kernel(dst_hbm, out_hbm, idx_v, ones_v, zero_v, acc_sh):
    c = lax.axis_index("c")
    s = lax.axis_index("s")
    wid = s * NC + c
    _fill_ones(ones_v, CHUNK)
    _zero_vmem(zero_v, ZR, DEGW)
    pltpu.sync_copy(dst_hbm.at[wid], idx_v)
    _zero_shared(zero_v, acc_sh, s)
    plsc.subcore_barrier()

    def body(j, _):
      pltpu.sync_copy(ones_v, acc_sh.at[idx_v.at[j]], add=True)
      return 0

    lax.fori_loop(0, NCHUNK, body, 0)
    plsc.subcore_barrier()
    _drain_shared(acc_sh, out_hbm, c, s)

  return deg_kernel(dst3)


def _sc_agg(u_lo, u_hi, src3, dst3):
  @functools.partial(
      pl.kernel,
      out_type=(
          jax.ShapeDtypeStruct((NC * N, FH), jnp.float32),
          jax.ShapeDtypeStruct((NC * N, FH), jnp.float32),
      ),
      mesh=_mesh(),
      compiler_params=pltpu.CompilerParams(use_tc_tiling_on_sc=False),
      scratch_types=[
          pltpu.VMEM((NCHUNK, CHUNK), jnp.int32),
          pltpu.VMEM((NCHUNK, CHUNK), jnp.int32),
          pltpu.VMEM((CHUNK, FH), jnp.float32),
          pltpu.VMEM((CHUNK, FH), jnp.float32),
          pltpu.VMEM((ZR, FH), jnp.float32),
          pltpu.VMEM_SHARED((N, FH), jnp.float32),
          pltpu.SemaphoreType.DMA,
          pltpu.SemaphoreType.DMA,
      ],
  )
  def agg_kernel(ulo_hbm, uhi_hbm, src_hbm, dst_hbm, olo_hbm, ohi_hbm,
                 src_v, dst_v, rows_a, rows_b, zero_v, acc_sh, sem_a, sem_b):
    c = lax.axis_index("c")
    s = lax.axis_index("s")
    wid = s * NC + c
    _zero_vmem(zero_v, ZR, FH)
    pltpu.sync_copy(src_hbm.at[wid], src_v)
    pltpu.sync_copy(dst_hbm.at[wid], dst_v)

    for u_hbm, out_hbm in ((ulo_hbm, olo_hbm), (uhi_hbm, ohi_hbm)):
      _zero_shared(zero_v, acc_sh, s)
      plsc.subcore_barrier()

      def body(j, _, u_hbm=u_hbm):
        pltpu.async_copy(u_hbm.at[src_v.at[j]], rows_a, sem_a).wait()
        pltpu.sync_copy(rows_a, acc_sh.at[dst_v.at[j]], add=True)
        return 0

      lax.fori_loop(0, NCHUNK, body, 0)
      plsc.subcore_barrier()
      _drain_shared(acc_sh, out_hbm, c, s)
      plsc.subcore_barrier()

  return agg_kernel(u_lo, u_hi, src3, dst3)


def _dinv_block(dp0, dp1):
  deg = dp0[:, 0:1] + dp1[:, 0:1] + 1.0
  return lax.rsqrt(deg)


def _tc_matmul_scale_body(x_ref, w_ref, dp0_ref, dp1_ref, ulo_ref, uhi_ref):
  dinv = _dinv_block(dp0_ref[...], dp1_ref[...])
  xw = jnp.dot(x_ref[...], w_ref[...], preferred_element_type=jnp.float32)
  u = xw * dinv
  ulo_ref[...] = u[:, :FH]
  uhi_ref[...] = u[:, FH:]


def _tc_mid_body(alo0_ref, alo1_ref, ahi0_ref, ahi1_ref, ulo_ref, uhi_ref,
                 dp0_ref, dp1_ref, b_ref, vlo_ref, vhi_ref):
  dinv = _dinv_block(dp0_ref[...], dp1_ref[...])
  t_lo = dinv * (alo0_ref[...] + alo1_ref[...] + ulo_ref[...]) + b_ref[:, :FH]
  t_hi = dinv * (ahi0_ref[...] + ahi1_ref[...] + uhi_ref[...]) + b_ref[:, FH:]
  vlo_ref[...] = dinv * jnp.maximum(t_lo, 0.0)
  vhi_ref[...] = dinv * jnp.maximum(t_hi, 0.0)


def _tc_final_body(alo0_ref, alo1_ref, ahi0_ref, ahi1_ref, vlo_ref, vhi_ref,
                   dp0_ref, dp1_ref, w_ref, b_ref, o_ref):
  dinv = _dinv_block(dp0_ref[...], dp1_ref[...])
  t_lo = dinv * (alo0_ref[...] + alo1_ref[...] + vlo_ref[...])
  t_hi = dinv * (ahi0_ref[...] + ahi1_ref[...] + vhi_ref[...])
  t = jnp.concatenate([t_lo, t_hi], axis=1)
  o_ref[...] = (
      jnp.dot(t, w_ref[...], preferred_element_type=jnp.float32) + b_ref[...])


def _row_spec(width):
  return pl.BlockSpec((ROWS_BLK, width), lambda i: (i, 0))


def _row_spec_hi(width):
  return pl.BlockSpec((ROWS_BLK, width), lambda i: (i + TC_GRID, 0))


def _full_spec(shape):
  return pl.BlockSpec(shape, lambda i: tuple(0 for _ in shape))


def _tc_matmul_scale(x, W, degp):
  return pl.pallas_call(
      _tc_matmul_scale_body,
      grid=(TC_GRID,),
      in_specs=[
          _row_spec(F),
          _full_spec((F, F)),
          _row_spec(DEGW),
          _row_spec_hi(DEGW),
      ],
      out_specs=(_row_spec(FH), _row_spec(FH)),
      out_shape=(
          jax.ShapeDtypeStruct((N, FH), jnp.float32),
          jax.ShapeDtypeStruct((N, FH), jnp.float32),
      ),
  )(x, W, degp, degp)


def _tc_mid(a_lo, a_hi, u_lo, u_hi, degp, b1):
  return pl.pallas_call(
      _tc_mid_body,
      grid=(TC_GRID,),
      in_specs=[
          _row_spec(FH),
          _row_spec_hi(FH),
          _row_spec(FH),
          _row_spec_hi(FH),
          _row_spec(FH),
          _row_spec(FH),
          _row_spec(DEGW),
          _row_spec_hi(DEGW),
          _full_spec((1, F)),
      ],
      out_specs=(_row_spec(FH), _row_spec(FH)),
      out_shape=(
          jax.ShapeDtypeStruct((N, FH), jnp.float32),
          jax.ShapeDtypeStruct((N, FH), jnp.float32),
      ),
  )(a_lo, a_lo, a_hi, a_hi, u_lo, u_hi, degp, degp, b1)


def _tc_final(a_lo, a_hi, v_lo, v_hi, degp, W, b2):
  return pl.pallas_call(
      _tc_final_body,
      grid=(TC_GRID,),
      in_specs=[
          _row_spec(FH),
          _row_spec_hi(FH),
          _row_spec(FH),
          _row_spec_hi(FH),
          _row_spec(FH),
          _row_spec(FH),
          _row_spec(DEGW),
          _row_spec_hi(DEGW),
          _full_spec((F, F)),
          _full_spec((1, F)),
      ],
      out_specs=_row_spec(F),
      out_shape=jax.ShapeDtypeStruct((N, F), jnp.float32),
  )(a_lo, a_lo, a_hi, a_hi, v_lo, v_hi, degp, degp, W, b2)


@jax.jit
def kernel(x, edge_index, W1, b1, W2, b2):
  src3 = edge_index[0].astype(jnp.int32).reshape(NW, NCHUNK, CHUNK)
  dst3 = edge_index[1].astype(jnp.int32).reshape(NW, NCHUNK, CHUNK)
  b1r = b1.reshape(1, F)
  b2r = b2.reshape(1, F)

  degp = _sc_degree(dst3)
  u_lo, u_hi = _tc_matmul_scale(x, W1, degp)
  a_lo, a_hi = _sc_agg(u_lo, u_hi, src3, dst3)
  v_lo, v_hi = _tc_mid(a_lo, a_hi, u_lo, u_hi, degp, b1r)
  b_lo, b_hi = _sc_agg(v_lo, v_hi, src3, dst3)
  return _tc_final(b_lo, b_hi, v_lo, v_hi, degp, W2, b2r)


# speedup vs baseline: 20.3954x; 1.4722x over previous
import functools
import jax
import jax.numpy as jnp
from jax import lax
from jax.experimental import pallas as pl
from jax.experimental.pallas import tpu as pltpu
from jax.experimental.pallas import tpu_sc as plsc

N = 10000
E = 320000
F = 128
FH = F // 2
NC = 2
NS = 16
NW = NC * NS
EPW = E // NW
CHUNK = 80
NCHUNK = EPW // CHUNK
KDEPTH = 5
DR = 624
ZR = 208
TAIL = N - NS * DR
DEGW = 16

ROWS_BLK = 1000
TC_GRID = N // ROWS_BLK


def _zero_vmem(ref, nrows, width):
  per_row = width // 16

  def body(i, _):
    r = i // per_row
    k = i % per_row
    ref[r, pl.ds(k * 16, 16)] = jnp.zeros((16,), jnp.float32)
    return 0

  lax.fori_loop(0, nrows * per_row, body, 0)


def _fill_ones(ref, nrows):
  def body(i, _):
    ref[i, :] = jnp.ones((16,), jnp.float32)
    return 0

  lax.fori_loop(0, nrows, body, 0)


def _zero_shared(zero_v, acc_sh, s):
  def body(i, _):
    pltpu.sync_copy(zero_v, acc_sh.at[pl.ds(s * DR + i * ZR, ZR)])
    return 0

  lax.fori_loop(0, DR // ZR, body, 0)

  @pl.when(s == NS - 1)
  def _():
    pltpu.sync_copy(zero_v.at[pl.ds(0, TAIL)], acc_sh.at[pl.ds(NS * DR, TAIL)])


def _drain_shared(acc_sh, out_hbm, c, s):
  pltpu.sync_copy(
      acc_sh.at[pl.ds(s * DR, DR)], out_hbm.at[pl.ds(c * N + s * DR, DR)])

  @pl.when(s == NS - 1)
  def _():
    pltpu.sync_copy(
        acc_sh.at[pl.ds(NS * DR, TAIL)],
        out_hbm.at[pl.ds(c * N + NS * DR, TAIL)])


def _mesh():
  return plsc.VectorSubcoreMesh(
      core_axis_name="c", subcore_axis_name="s", num_cores=NC,
      num_subcores=NS)


def _sc_degree(dst3):
  @functools.partial(
      pl.kernel,
      out_type=jax.ShapeDtypeStruct((NC * N, DEGW), jnp.float32),
      mesh=_mesh(),
      compiler_params=pltpu.CompilerParams(use_tc_tiling_on_sc=False),
      scratch_types=[
          pltpu.VMEM((NCHUNK, CHUNK), jnp.int32),
          pltpu.VMEM((CHUNK, DEGW), jnp.float32),
          pltpu.VMEM((ZR, DEGW), jnp.float32),
          pltpu.VMEM_SHARED((N, DEGW), jnp.float32),
      ],
  )
  def deg_kernel(dst_hbm, out_hbm, idx_v, ones_v, zero_v, acc_sh):
    c = lax.axis_index("c")
    s = lax.axis_index("s")
    wid = s * NC + c
    _fill_ones(ones_v, CHUNK)
    _zero_vmem(zero_v, ZR, DEGW)
    pltpu.sync_copy(dst_hbm.at[wid], idx_v)
    _zero_shared(zero_v, acc_sh, s)
    plsc.subcore_barrier()

    def body(j, _):
      pltpu.sync_copy(ones_v, acc_sh.at[idx_v.at[j]], add=True)
      return 0

    lax.fori_loop(0, NCHUNK, body, 0)
    plsc.subcore_barrier()
    _drain_shared(acc_sh, out_hbm, c, s)

  return deg_kernel(dst3)


def _sc_agg(u_lo, u_hi, src3, dst3):
  @functools.partial(
      pl.kernel,
      out_type=(
          jax.ShapeDtypeStruct((NC * N, FH), jnp.float32),
          jax.ShapeDtypeStruct((NC * N, FH), jnp.float32),
      ),
      mesh=_mesh(),
      compiler_params=pltpu.CompilerParams(use_tc_tiling_on_sc=False),
      scratch_types=[
          pltpu.VMEM((NCHUNK, CHUNK), jnp.int32),
          pltpu.VMEM((NCHUNK, CHUNK), jnp.int32),
          pltpu.VMEM((CHUNK, FH), jnp.float32),
          pltpu.VMEM((CHUNK, FH), jnp.float32),
          pltpu.VMEM((CHUNK, FH), jnp.float32),
          pltpu.VMEM((CHUNK, FH), jnp.float32),
          pltpu.VMEM((CHUNK, FH), jnp.float32),
          pltpu.VMEM((ZR, FH), jnp.float32),
          pltpu.VMEM_SHARED((N, FH), jnp.float32),
          pltpu.SemaphoreType.DMA,
          pltpu.SemaphoreType.DMA,
      ],
  )
  def agg_kernel(ulo_hbm, uhi_hbm, src_hbm, dst_hbm, olo_hbm, ohi_hbm,
                 src_v, dst_v, rows0, rows1, rows2, rows3, rows4,
                 zero_v, acc_sh, sem_g, sem_s):
    rows = (rows0, rows1, rows2, rows3, rows4)
    c = lax.axis_index("c")
    s = lax.axis_index("s")
    wid = s * NC + c
    _zero_vmem(zero_v, ZR, FH)
    pltpu.sync_copy(src_hbm.at[wid], src_v)
    pltpu.sync_copy(dst_hbm.at[wid], dst_v)

    for u_hbm, out_hbm in ((ulo_hbm, olo_hbm), (uhi_hbm, ohi_hbm)):
      _zero_shared(zero_v, acc_sh, s)
      plsc.subcore_barrier()

      def body(g, _, u_hbm=u_hbm):
        base = g * KDEPTH
        gd = [
            pltpu.async_copy(
                u_hbm.at[src_v.at[base + k]], rows[k], sem_g)
            for k in range(KDEPTH)
        ]
        for d in gd:
          d.wait()
        for k in range(KDEPTH):
          pltpu.sync_copy(rows[k], acc_sh.at[dst_v.at[base + k]], add=True)
        return 0

      lax.fori_loop(0, NCHUNK // KDEPTH, body, 0)
      plsc.subcore_barrier()
      _drain_shared(acc_sh, out_hbm, c, s)
      plsc.subcore_barrier()

  return agg_kernel(u_lo, u_hi, src3, dst3)


def _dinv_block(dp0, dp1):
  deg = dp0[:, 0:1] + dp1[:, 0:1] + 1.0
  return lax.rsqrt(deg)


def _tc_matmul_scale_body(x_ref, w_ref, dp0_ref, dp1_ref, ulo_ref, uhi_ref):
  dinv = _dinv_block(dp0_ref[...], dp1_ref[...])
  xw = jnp.dot(x_ref[...], w_ref[...], preferred_element_type=jnp.float32)
  u = xw * dinv
  ulo_ref[...] = u[:, :FH]
  uhi_ref[...] = u[:, FH:]


def _tc_mid_body(alo0_ref, alo1_ref, ahi0_ref, ahi1_ref, ulo_ref, uhi_ref,
                 dp0_ref, dp1_ref, b_ref, vlo_ref, vhi_ref):
  dinv = _dinv_block(dp0_ref[...], dp1_ref[...])
  t_lo = dinv * (alo0_ref[...] + alo1_ref[...] + ulo_ref[...]) + b_ref[:, :FH]
  t_hi = dinv * (ahi0_ref[...] + ahi1_ref[...] + uhi_ref[...]) + b_ref[:, FH:]
  vlo_ref[...] = dinv * jnp.maximum(t_lo, 0.0)
  vhi_ref[...] = dinv * jnp.maximum(t_hi, 0.0)


def _tc_final_body(alo0_ref, alo1_ref, ahi0_ref, ahi1_ref, vlo_ref, vhi_ref,
                   dp0_ref, dp1_ref, w_ref, b_ref, o_ref):
  dinv = _dinv_block(dp0_ref[...], dp1_ref[...])
  t_lo = dinv * (alo0_ref[...] + alo1_ref[...] + vlo_ref[...])
  t_hi = dinv * (ahi0_ref[...] + ahi1_ref[...] + vhi_ref[...])
  t = jnp.concatenate([t_lo, t_hi], axis=1)
  o_ref[...] = (
      jnp.dot(t, w_ref[...], preferred_element_type=jnp.float32) + b_ref[...])


def _row_spec(width):
  return pl.BlockSpec((ROWS_BLK, width), lambda i: (i, 0))


def _row_spec_hi(width):
  return pl.BlockSpec((ROWS_BLK, width), lambda i: (i + TC_GRID, 0))


def _full_spec(shape):
  return pl.BlockSpec(shape, lambda i: tuple(0 for _ in shape))


def _tc_matmul_scale(x, W, degp):
  return pl.pallas_call(
      _tc_matmul_scale_body,
      grid=(TC_GRID,),
      in_specs=[
          _row_spec(F),
          _full_spec((F, F)),
          _row_spec(DEGW),
          _row_spec_hi(DEGW),
      ],
      out_specs=(_row_spec(FH), _row_spec(FH)),
      out_shape=(
          jax.ShapeDtypeStruct((N, FH), jnp.float32),
          jax.ShapeDtypeStruct((N, FH), jnp.float32),
      ),
  )(x, W, degp, degp)


def _tc_mid(a_lo, a_hi, u_lo, u_hi, degp, b1):
  return pl.pallas_call(
      _tc_mid_body,
      grid=(TC_GRID,),
      in_specs=[
          _row_spec(FH),
          _row_spec_hi(FH),
          _row_spec(FH),
          _row_spec_hi(FH),
          _row_spec(FH),
          _row_spec(FH),
          _row_spec(DEGW),
          _row_spec_hi(DEGW),
          _full_spec((1, F)),
      ],
      out_specs=(_row_spec(FH), _row_spec(FH)),
      out_shape=(
          jax.ShapeDtypeStruct((N, FH), jnp.float32),
          jax.ShapeDtypeStruct((N, FH), jnp.float32),
      ),
  )(a_lo, a_lo, a_hi, a_hi, u_lo, u_hi, degp, degp, b1)


def _tc_final(a_lo, a_hi, v_lo, v_hi, degp, W, b2):
  return pl.pallas_call(
      _tc_final_body,
      grid=(TC_GRID,),
      in_specs=[
          _row_spec(FH),
          _row_spec_hi(FH),
          _row_spec(FH),
          _row_spec_hi(FH),
          _row_spec(FH),
          _row_spec(FH),
          _row_spec(DEGW),
          _row_spec_hi(DEGW),
          _full_spec((F, F)),
          _full_spec((1, F)),
      ],
      out_specs=_row_spec(F),
      out_shape=jax.ShapeDtypeStruct((N, F), jnp.float32),
  )(a_lo, a_lo, a_hi, a_hi, v_lo, v_hi, degp, degp, W, b2)


@jax.jit
def kernel(x, edge_index, W1, b1, W2, b2):
  src3 = edge_index[0].astype(jnp.int32).reshape(NW, NCHUNK, CHUNK)
  dst3 = edge_index[1].astype(jnp.int32).reshape(NW, NCHUNK, CHUNK)
  b1r = b1.reshape(1, F)
  b2r = b2.reshape(1, F)

  degp = _sc_degree(dst3)
  u_lo, u_hi = _tc_matmul_scale(x, W1, degp)
  a_lo, a_hi = _sc_agg(u_lo, u_hi, src3, dst3)
  v_lo, v_hi = _tc_mid(a_lo, a_hi, u_lo, u_hi, degp, b1r)
  b_lo, b_hi = _sc_agg(v_lo, v_hi, src3, dst3)
  return _tc_final(b_lo, b_hi, v_lo, v_hi, degp, W2, b2r)


# speedup vs baseline: 21.5253x; 1.0554x over previous
import functools
import jax
import jax.numpy as jnp
from jax import lax
from jax.experimental import pallas as pl
from jax.experimental.pallas import tpu as pltpu
from jax.experimental.pallas import tpu_sc as plsc

N = 10000
E = 320000
F = 128
FH = F // 2
NC = 2
NS = 16
NW = NC * NS
EPW = E // NW
CHUNK = 80
NCHUNK = EPW // CHUNK
KDEPTH = 5
DR = 624
ZR = 208
TAIL = N - NS * DR
DEGW = 16

ROWS_BLK = 1000
TC_GRID = N // ROWS_BLK


def _zero_vmem(ref, nrows, width):
  per_row = width // 16

  def body(i, _):
    r = i // per_row
    k = i % per_row
    ref[r, pl.ds(k * 16, 16)] = jnp.zeros((16,), jnp.float32)
    return 0

  lax.fori_loop(0, nrows * per_row, body, 0)


def _fill_ones(ref, nrows):
  def body(i, _):
    ref[i, :] = jnp.ones((16,), jnp.float32)
    return 0

  lax.fori_loop(0, nrows, body, 0)


def _zero_shared(zero_v, acc_sh, s):
  def body(i, _):
    pltpu.sync_copy(zero_v, acc_sh.at[pl.ds(s * DR + i * ZR, ZR)])
    return 0

  lax.fori_loop(0, DR // ZR, body, 0)

  @pl.when(s == NS - 1)
  def _():
    pltpu.sync_copy(zero_v.at[pl.ds(0, TAIL)], acc_sh.at[pl.ds(NS * DR, TAIL)])


def _drain_shared(acc_sh, out_hbm, c, s):
  pltpu.sync_copy(
      acc_sh.at[pl.ds(s * DR, DR)], out_hbm.at[pl.ds(c * N + s * DR, DR)])

  @pl.when(s == NS - 1)
  def _():
    pltpu.sync_copy(
        acc_sh.at[pl.ds(NS * DR, TAIL)],
        out_hbm.at[pl.ds(c * N + NS * DR, TAIL)])


def _mesh():
  return plsc.VectorSubcoreMesh(
      core_axis_name="c", subcore_axis_name="s", num_cores=NC,
      num_subcores=NS)


def _sc_degree(dst3):
  @functools.partial(
      pl.kernel,
      out_type=jax.ShapeDtypeStruct((NC * N, DEGW), jnp.float32),
      mesh=_mesh(),
      compiler_params=pltpu.CompilerParams(use_tc_tiling_on_sc=False),
      scratch_types=[
          pltpu.VMEM((NCHUNK, CHUNK), jnp.int32),
          pltpu.VMEM((CHUNK, DEGW), jnp.float32),
          pltpu.VMEM((ZR, DEGW), jnp.float32),
          pltpu.VMEM_SHARED((N, DEGW), jnp.float32),
      ],
  )
  def deg_kernel(dst_hbm, out_hbm, idx_v, ones_v, zero_v, acc_sh):
    c = lax.axis_index("c")
    s = lax.axis_index("s")
    wid = s * NC + c
    _fill_ones(ones_v, CHUNK)
    _zero_vmem(zero_v, ZR, DEGW)
    pltpu.sync_copy(dst_hbm.at[wid], idx_v)
    _zero_shared(zero_v, acc_sh, s)
    plsc.subcore_barrier()

    def body(j, _):
      pltpu.sync_copy(ones_v, acc_sh.at[idx_v.at[j]], add=True)
      return 0

    lax.fori_loop(0, NCHUNK, body, 0)
    plsc.subcore_barrier()
    _drain_shared(acc_sh, out_hbm, c, s)

  return deg_kernel(dst3)


def _sc_agg(u_lo, u_hi, src3, dst3):
  @functools.partial(
      pl.kernel,
      out_type=(
          jax.ShapeDtypeStruct((NC * N, FH), jnp.float32),
          jax.ShapeDtypeStruct((NC * N, FH), jnp.float32),
      ),
      mesh=_mesh(),
      compiler_params=pltpu.CompilerParams(use_tc_tiling_on_sc=False),
      scratch_types=[
          pltpu.VMEM((NCHUNK, CHUNK), jnp.int32),
          pltpu.VMEM((NCHUNK, CHUNK), jnp.int32),
          pltpu.VMEM((CHUNK, FH), jnp.float32),
          pltpu.VMEM((CHUNK, FH), jnp.float32),
          pltpu.VMEM((CHUNK, FH), jnp.float32),
          pltpu.VMEM((CHUNK, FH), jnp.float32),
          pltpu.VMEM((CHUNK, FH), jnp.float32),
          pltpu.VMEM((ZR, FH), jnp.float32),
          pltpu.VMEM_SHARED((N, FH), jnp.float32),
          pltpu.SemaphoreType.DMA,
          pltpu.SemaphoreType.DMA,
      ],
  )
  def agg_kernel(ulo_hbm, uhi_hbm, src_hbm, dst_hbm, olo_hbm, ohi_hbm,
                 src_v, dst_v, rows0, rows1, rows2, rows3, rows4,
                 zero_v, acc_sh, sem_g, sem_s):
    rows = (rows0, rows1, rows2, rows3, rows4)
    c = lax.axis_index("c")
    s = lax.axis_index("s")
    wid = s * NC + c
    _zero_vmem(zero_v, ZR, FH)
    pltpu.sync_copy(src_hbm.at[wid], src_v)
    pltpu.sync_copy(dst_hbm.at[wid], dst_v)

    for u_hbm, out_hbm in ((ulo_hbm, olo_hbm), (uhi_hbm, ohi_hbm)):
      _zero_shared(zero_v, acc_sh, s)
      plsc.subcore_barrier()

      def body(g, _, u_hbm=u_hbm):
        base = g * KDEPTH
        gd = [
            pltpu.async_copy(
                u_hbm.at[src_v.at[base + k]], rows[k], sem_g)
            for k in range(KDEPTH)
        ]
        for d in gd:
          d.wait()
        sd = [
            pltpu.async_copy(
                rows[k], acc_sh.at[dst_v.at[base + k]], sem_s, add=True)
            for k in range(KDEPTH)
        ]
        for d in sd:
          d.wait()
        return 0

      lax.fori_loop(0, NCHUNK // KDEPTH, body, 0)
      plsc.subcore_barrier()
      _drain_shared(acc_sh, out_hbm, c, s)
      plsc.subcore_barrier()

  return agg_kernel(u_lo, u_hi, src3, dst3)


def _dinv_block(dp0, dp1):
  deg = dp0[:, 0:1] + dp1[:, 0:1] + 1.0
  return lax.rsqrt(deg)


def _tc_matmul_scale_body(x_ref, w_ref, dp0_ref, dp1_ref, ulo_ref, uhi_ref):
  dinv = _dinv_block(dp0_ref[...], dp1_ref[...])
  xw = jnp.dot(x_ref[...], w_ref[...], preferred_element_type=jnp.float32)
  u = xw * dinv
  ulo_ref[...] = u[:, :FH]
  uhi_ref[...] = u[:, FH:]


def _tc_mid_body(alo0_ref, alo1_ref, ahi0_ref, ahi1_ref, ulo_ref, uhi_ref,
                 dp0_ref, dp1_ref, b_ref, vlo_ref, vhi_ref):
  dinv = _dinv_block(dp0_ref[...], dp1_ref[...])
  t_lo = dinv * (alo0_ref[...] + alo1_ref[...] + ulo_ref[...]) + b_ref[:, :FH]
  t_hi = dinv * (ahi0_ref[...] + ahi1_ref[...] + uhi_ref[...]) + b_ref[:, FH:]
  vlo_ref[...] = dinv * jnp.maximum(t_lo, 0.0)
  vhi_ref[...] = dinv * jnp.maximum(t_hi, 0.0)


def _tc_final_body(alo0_ref, alo1_ref, ahi0_ref, ahi1_ref, vlo_ref, vhi_ref,
                   dp0_ref, dp1_ref, w_ref, b_ref, o_ref):
  dinv = _dinv_block(dp0_ref[...], dp1_ref[...])
  t_lo = dinv * (alo0_ref[...] + alo1_ref[...] + vlo_ref[...])
  t_hi = dinv * (ahi0_ref[...] + ahi1_ref[...] + vhi_ref[...])
  t = jnp.concatenate([t_lo, t_hi], axis=1)
  o_ref[...] = (
      jnp.dot(t, w_ref[...], preferred_element_type=jnp.float32) + b_ref[...])


def _row_spec(width):
  return pl.BlockSpec((ROWS_BLK, width), lambda i: (i, 0))


def _row_spec_hi(width):
  return pl.BlockSpec((ROWS_BLK, width), lambda i: (i + TC_GRID, 0))


def _full_spec(shape):
  return pl.BlockSpec(shape, lambda i: tuple(0 for _ in shape))


def _tc_matmul_scale(x, W, degp):
  return pl.pallas_call(
      _tc_matmul_scale_body,
      grid=(TC_GRID,),
      in_specs=[
          _row_spec(F),
          _full_spec((F, F)),
          _row_spec(DEGW),
          _row_spec_hi(DEGW),
      ],
      out_specs=(_row_spec(FH), _row_spec(FH)),
      out_shape=(
          jax.ShapeDtypeStruct((N, FH), jnp.float32),
          jax.ShapeDtypeStruct((N, FH), jnp.float32),
      ),
  )(x, W, degp, degp)


def _tc_mid(a_lo, a_hi, u_lo, u_hi, degp, b1):
  return pl.pallas_call(
      _tc_mid_body,
      grid=(TC_GRID,),
      in_specs=[
          _row_spec(FH),
          _row_spec_hi(FH),
          _row_spec(FH),
          _row_spec_hi(FH),
          _row_spec(FH),
          _row_spec(FH),
          _row_spec(DEGW),
          _row_spec_hi(DEGW),
          _full_spec((1, F)),
      ],
      out_specs=(_row_spec(FH), _row_spec(FH)),
      out_shape=(
          jax.ShapeDtypeStruct((N, FH), jnp.float32),
          jax.ShapeDtypeStruct((N, FH), jnp.float32),
      ),
  )(a_lo, a_lo, a_hi, a_hi, u_lo, u_hi, degp, degp, b1)


def _tc_final(a_lo, a_hi, v_lo, v_hi, degp, W, b2):
  return pl.pallas_call(
      _tc_final_body,
      grid=(TC_GRID,),
      in_specs=[
          _row_spec(FH),
          _row_spec_hi(FH),
          _row_spec(FH),
          _row_spec_hi(FH),
          _row_spec(FH),
          _row_spec(FH),
          _row_spec(DEGW),
          _row_spec_hi(DEGW),
          _full_spec((F, F)),
          _full_spec((1, F)),
      ],
      out_specs=_row_spec(F),
      out_shape=jax.ShapeDtypeStruct((N, F), jnp.float32),
  )(a_lo, a_lo, a_hi, a_hi, v_lo, v_hi, degp, degp, W, b2)


@jax.jit
def kernel(x, edge_index, W1, b1, W2, b2):
  src3 = edge_index[0].astype(jnp.int32).reshape(NW, NCHUNK, CHUNK)
  dst3 = edge_index[1].astype(jnp.int32).reshape(NW, NCHUNK, CHUNK)
  b1r = b1.reshape(1, F)
  b2r = b2.reshape(1, F)

  degp = _sc_degree(dst3)
  u_lo, u_hi = _tc_matmul_scale(x, W1, degp)
  a_lo, a_hi = _sc_agg(u_lo, u_hi, src3, dst3)
  v_lo, v_hi = _tc_mid(a_lo, a_hi, u_lo, u_hi, degp, b1r)
  b_lo, b_hi = _sc_agg(v_lo, v_hi, src3, dst3)
  return _tc_final(b_lo, b_hi, v_lo, v_hi, degp, W2, b2r)


# speedup vs baseline: 22.4326x; 1.0422x over previous
import functools
import jax
import jax.numpy as jnp
from jax import lax
from jax.experimental import pallas as pl
from jax.experimental.pallas import tpu as pltpu
from jax.experimental.pallas import tpu_sc as plsc

N = 10000
E = 320000
F = 128
FH = F // 2
NC = 2
NS = 16
NW = NC * NS
EPW = E // NW
CHUNK = 80
NCHUNK = EPW // CHUNK
KDEPTH = 10
PEEL = NCHUNK % KDEPTH
DR = 624
ZR = 208
TAIL = N - NS * DR
DEGW = 16

ROWS_BLK = 1000
TC_GRID = N // ROWS_BLK


def _zero_vmem(ref, nrows, width):
  per_row = width // 16

  def body(i, _):
    r = i // per_row
    k = i % per_row
    ref[r, pl.ds(k * 16, 16)] = jnp.zeros((16,), jnp.float32)
    return 0

  lax.fori_loop(0, nrows * per_row, body, 0)


def _fill_ones(ref, nrows):
  def body(i, _):
    ref[i, :] = jnp.ones((16,), jnp.float32)
    return 0

  lax.fori_loop(0, nrows, body, 0)


def _zero_shared(zero_v, acc_sh, s):
  def body(i, _):
    pltpu.sync_copy(zero_v, acc_sh.at[pl.ds(s * DR + i * ZR, ZR)])
    return 0

  lax.fori_loop(0, DR // ZR, body, 0)

  @pl.when(s == NS - 1)
  def _():
    pltpu.sync_copy(zero_v.at[pl.ds(0, TAIL)], acc_sh.at[pl.ds(NS * DR, TAIL)])


def _drain_shared(acc_sh, out_hbm, c, s):
  pltpu.sync_copy(
      acc_sh.at[pl.ds(s * DR, DR)], out_hbm.at[pl.ds(c * N + s * DR, DR)])

  @pl.when(s == NS - 1)
  def _():
    pltpu.sync_copy(
        acc_sh.at[pl.ds(NS * DR, TAIL)],
        out_hbm.at[pl.ds(c * N + NS * DR, TAIL)])


def _mesh():
  return plsc.VectorSubcoreMesh(
      core_axis_name="c", subcore_axis_name="s", num_cores=NC,
      num_subcores=NS)


def _sc_degree(dst3):
  @functools.partial(
      pl.kernel,
      out_type=jax.ShapeDtypeStruct((NC * N, DEGW), jnp.float32),
      mesh=_mesh(),
      compiler_params=pltpu.CompilerParams(use_tc_tiling_on_sc=False),
      scratch_types=[
          pltpu.VMEM((NCHUNK, CHUNK), jnp.int32),
          pltpu.VMEM((CHUNK, DEGW), jnp.float32),
          pltpu.VMEM((ZR, DEGW), jnp.float32),
          pltpu.VMEM_SHARED((N, DEGW), jnp.float32),
      ],
  )
  def deg_kernel(dst_hbm, out_hbm, idx_v, ones_v, zero_v, acc_sh):
    c = lax.axis_index("c")
    s = lax.axis_index("s")
    wid = s * NC + c
    _fill_ones(ones_v, CHUNK)
    _zero_vmem(zero_v, ZR, DEGW)
    pltpu.sync_copy(dst_hbm.at[wid], idx_v)
    _zero_shared(zero_v, acc_sh, s)
    plsc.subcore_barrier()

    def body(j, _):
      pltpu.sync_copy(ones_v, acc_sh.at[idx_v.at[j]], add=True)
      return 0

    lax.fori_loop(0, NCHUNK, body, 0)
    plsc.subcore_barrier()
    _drain_shared(acc_sh, out_hbm, c, s)

  return deg_kernel(dst3)


def _sc_agg(u_lo, u_hi, src3, dst3):
  @functools.partial(
      pl.kernel,
      out_type=(
          jax.ShapeDtypeStruct((NC * N, FH), jnp.float32),
          jax.ShapeDtypeStruct((NC * N, FH), jnp.float32),
      ),
      mesh=_mesh(),
      compiler_params=pltpu.CompilerParams(use_tc_tiling_on_sc=False),
      scratch_types=[
          pltpu.VMEM((NCHUNK, CHUNK), jnp.int32),
          pltpu.VMEM((NCHUNK, CHUNK), jnp.int32),
          [pltpu.VMEM((CHUNK, FH), jnp.float32)] * KDEPTH,
          pltpu.VMEM((ZR, FH), jnp.float32),
          pltpu.VMEM_SHARED((N, FH), jnp.float32),
          pltpu.SemaphoreType.DMA,
          pltpu.SemaphoreType.DMA,
      ],
  )
  def agg_kernel(ulo_hbm, uhi_hbm, src_hbm, dst_hbm, olo_hbm, ohi_hbm,
                 src_v, dst_v, rows, zero_v, acc_sh, sem_g, sem_s):
    c = lax.axis_index("c")
    s = lax.axis_index("s")
    wid = s * NC + c
    _zero_vmem(zero_v, ZR, FH)
    pltpu.sync_copy(src_hbm.at[wid], src_v)
    pltpu.sync_copy(dst_hbm.at[wid], dst_v)

    def run_group(u_hbm, base, nk):
      gd = [
          pltpu.async_copy(u_hbm.at[src_v.at[base + k]], rows[k], sem_g)
          for k in range(nk)
      ]
      for d in gd:
        d.wait()
      sd = [
          pltpu.async_copy(
              rows[k], acc_sh.at[dst_v.at[base + k]], sem_s, add=True)
          for k in range(nk)
      ]
      for d in sd:
        d.wait()

    for u_hbm, out_hbm in ((ulo_hbm, olo_hbm), (uhi_hbm, ohi_hbm)):
      _zero_shared(zero_v, acc_sh, s)
      plsc.subcore_barrier()

      def body(g, _, u_hbm=u_hbm):
        run_group(u_hbm, g * KDEPTH, KDEPTH)
        return 0

      lax.fori_loop(0, NCHUNK // KDEPTH, body, 0)
      run_group(u_hbm, (NCHUNK // KDEPTH) * KDEPTH, PEEL)
      plsc.subcore_barrier()
      _drain_shared(acc_sh, out_hbm, c, s)
      plsc.subcore_barrier()

  return agg_kernel(u_lo, u_hi, src3, dst3)


def _dinv_block(dp0, dp1):
  deg = dp0[:, 0:1] + dp1[:, 0:1] + 1.0
  return lax.rsqrt(deg)


def _tc_matmul_scale_body(x_ref, w_ref, dp0_ref, dp1_ref, ulo_ref, uhi_ref):
  dinv = _dinv_block(dp0_ref[...], dp1_ref[...])
  xw = jnp.dot(x_ref[...], w_ref[...], preferred_element_type=jnp.float32)
  u = xw * dinv
  ulo_ref[...] = u[:, :FH]
  uhi_ref[...] = u[:, FH:]


def _tc_mid_body(alo0_ref, alo1_ref, ahi0_ref, ahi1_ref, ulo_ref, uhi_ref,
                 dp0_ref, dp1_ref, b_ref, vlo_ref, vhi_ref):
  dinv = _dinv_block(dp0_ref[...], dp1_ref[...])
  t_lo = dinv * (alo0_ref[...] + alo1_ref[...] + ulo_ref[...]) + b_ref[:, :FH]
  t_hi = dinv * (ahi0_ref[...] + ahi1_ref[...] + uhi_ref[...]) + b_ref[:, FH:]
  vlo_ref[...] = dinv * jnp.maximum(t_lo, 0.0)
  vhi_ref[...] = dinv * jnp.maximum(t_hi, 0.0)


def _tc_final_body(alo0_ref, alo1_ref, ahi0_ref, ahi1_ref, vlo_ref, vhi_ref,
                   dp0_ref, dp1_ref, w_ref, b_ref, o_ref):
  dinv = _dinv_block(dp0_ref[...], dp1_ref[...])
  t_lo = dinv * (alo0_ref[...] + alo1_ref[...] + vlo_ref[...])
  t_hi = dinv * (ahi0_ref[...] + ahi1_ref[...] + vhi_ref[...])
  t = jnp.concatenate([t_lo, t_hi], axis=1)
  o_ref[...] = (
      jnp.dot(t, w_ref[...], preferred_element_type=jnp.float32) + b_ref[...])


def _row_spec(width):
  return pl.BlockSpec((ROWS_BLK, width), lambda i: (i, 0))


def _row_spec_hi(width):
  return pl.BlockSpec((ROWS_BLK, width), lambda i: (i + TC_GRID, 0))


def _full_spec(shape):
  return pl.BlockSpec(shape, lambda i: tuple(0 for _ in shape))


def _tc_matmul_scale(x, W, degp):
  return pl.pallas_call(
      _tc_matmul_scale_body,
      grid=(TC_GRID,),
      in_specs=[
          _row_spec(F),
          _full_spec((F, F)),
          _row_spec(DEGW),
          _row_spec_hi(DEGW),
      ],
      out_specs=(_row_spec(FH), _row_spec(FH)),
      out_shape=(
          jax.ShapeDtypeStruct((N, FH), jnp.float32),
          jax.ShapeDtypeStruct((N, FH), jnp.float32),
      ),
  )(x, W, degp, degp)


def _tc_mid(a_lo, a_hi, u_lo, u_hi, degp, b1):
  return pl.pallas_call(
      _tc_mid_body,
      grid=(TC_GRID,),
      in_specs=[
          _row_spec(FH),
          _row_spec_hi(FH),
          _row_spec(FH),
          _row_spec_hi(FH),
          _row_spec(FH),
          _row_spec(FH),
          _row_spec(DEGW),
          _row_spec_hi(DEGW),
          _full_spec((1, F)),
      ],
      out_specs=(_row_spec(FH), _row_spec(FH)),
      out_shape=(
          jax.ShapeDtypeStruct((N, FH), jnp.float32),
          jax.ShapeDtypeStruct((N, FH), jnp.float32),
      ),
  )(a_lo, a_lo, a_hi, a_hi, u_lo, u_hi, degp, degp, b1)


def _tc_final(a_lo, a_hi, v_lo, v_hi, degp, W, b2):
  return pl.pallas_call(
      _tc_final_body,
      grid=(TC_GRID,),
      in_specs=[
          _row_spec(FH),
          _row_spec_hi(FH),
          _row_spec(FH),
          _row_spec_hi(FH),
          _row_spec(FH),
          _row_spec(FH),
          _row_spec(DEGW),
          _row_spec_hi(DEGW),
          _full_spec((F, F)),
          _full_spec((1, F)),
      ],
      out_specs=_row_spec(F),
      out_shape=jax.ShapeDtypeStruct((N, F), jnp.float32),
  )(a_lo, a_lo, a_hi, a_hi, v_lo, v_hi, degp, degp, W, b2)


@jax.jit
def kernel(x, edge_index, W1, b1, W2, b2):
  src3 = edge_index[0].astype(jnp.int32).reshape(NW, NCHUNK, CHUNK)
  dst3 = edge_index[1].astype(jnp.int32).reshape(NW, NCHUNK, CHUNK)
  b1r = b1.reshape(1, F)
  b2r = b2.reshape(1, F)

  degp = _sc_degree(dst3)
  u_lo, u_hi = _tc_matmul_scale(x, W1, degp)
  a_lo, a_hi = _sc_agg(u_lo, u_hi, src3, dst3)
  v_lo, v_hi = _tc_mid(a_lo, a_hi, u_lo, u_hi, degp, b1r)
  b_lo, b_hi = _sc_agg(v_lo, v_hi, src3, dst3)
  return _tc_final(b_lo, b_hi, v_lo, v_hi, degp, W2, b2r)


# speedup vs baseline: 22.7590x; 1.0145x over previous
import functools
import jax
import jax.numpy as jnp
from jax import lax
from jax.experimental import pallas as pl
from jax.experimental.pallas import tpu as pltpu
from jax.experimental.pallas import tpu_sc as plsc

N = 10000
E = 320000
F = 128
FH = F // 2
NC = 2
NS = 16
NW = NC * NS
EPW = E // NW
CHUNK = 80
NCHUNK = EPW // CHUNK
KDEPTH = 10
PEEL = NCHUNK % KDEPTH
DR = 624
ZR = 208
TAIL = N - NS * DR
DEGW = 16

ROWS_BLK = 1000
TC_GRID = N // ROWS_BLK


def _zero_vmem(ref, nrows, width):
  per_row = width // 16

  def body(i, _):
    r = i // per_row
    k = i % per_row
    ref[r, pl.ds(k * 16, 16)] = jnp.zeros((16,), jnp.float32)
    return 0

  lax.fori_loop(0, nrows * per_row, body, 0)


def _fill_ones(ref, nrows):
  def body(i, _):
    ref[i, :] = jnp.ones((16,), jnp.float32)
    return 0

  lax.fori_loop(0, nrows, body, 0)


def _zero_shared(zero_v, acc_sh, s):
  def body(i, _):
    pltpu.sync_copy(zero_v, acc_sh.at[pl.ds(s * DR + i * ZR, ZR)])
    return 0

  lax.fori_loop(0, DR // ZR, body, 0)

  @pl.when(s == NS - 1)
  def _():
    pltpu.sync_copy(zero_v.at[pl.ds(0, TAIL)], acc_sh.at[pl.ds(NS * DR, TAIL)])


def _drain_shared(acc_sh, out_hbm, c, s):
  pltpu.sync_copy(
      acc_sh.at[pl.ds(s * DR, DR)], out_hbm.at[pl.ds(c * N + s * DR, DR)])

  @pl.when(s == NS - 1)
  def _():
    pltpu.sync_copy(
        acc_sh.at[pl.ds(NS * DR, TAIL)],
        out_hbm.at[pl.ds(c * N + NS * DR, TAIL)])


def _mesh():
  return plsc.VectorSubcoreMesh(
      core_axis_name="c", subcore_axis_name="s", num_cores=NC,
      num_subcores=NS)


def _sc_degree(dst3):
  @functools.partial(
      pl.kernel,
      out_type=jax.ShapeDtypeStruct((NC * N, DEGW), jnp.float32),
      mesh=_mesh(),
      compiler_params=pltpu.CompilerParams(use_tc_tiling_on_sc=False),
      scratch_types=[
          pltpu.VMEM((NCHUNK, CHUNK), jnp.int32),
          pltpu.VMEM((CHUNK, DEGW), jnp.float32),
          pltpu.VMEM((ZR, DEGW), jnp.float32),
          pltpu.VMEM_SHARED((N, DEGW), jnp.float32),
          pltpu.SemaphoreType.DMA,
      ],
  )
  def deg_kernel(dst_hbm, out_hbm, idx_v, ones_v, zero_v, acc_sh, sem_s):
    c = lax.axis_index("c")
    s = lax.axis_index("s")
    wid = s * NC + c
    _fill_ones(ones_v, CHUNK)
    _zero_vmem(zero_v, ZR, DEGW)
    pltpu.sync_copy(dst_hbm.at[wid], idx_v)
    _zero_shared(zero_v, acc_sh, s)
    plsc.subcore_barrier()

    def ones_group(base, nk):
      sd = [
          pltpu.async_copy(ones_v, acc_sh.at[idx_v.at[base + k]], sem_s,
                           add=True)
          for k in range(nk)
      ]
      for d in sd:
        d.wait()

    def body(g, _):
      ones_group(g * KDEPTH, KDEPTH)
      return 0

    lax.fori_loop(0, NCHUNK // KDEPTH, body, 0)
    ones_group((NCHUNK // KDEPTH) * KDEPTH, PEEL)
    plsc.subcore_barrier()
    _drain_shared(acc_sh, out_hbm, c, s)

  return deg_kernel(dst3)


def _sc_agg(u_lo, u_hi, src3, dst3):
  @functools.partial(
      pl.kernel,
      out_type=(
          jax.ShapeDtypeStruct((NC * N, FH), jnp.float32),
          jax.ShapeDtypeStruct((NC * N, FH), jnp.float32),
      ),
      mesh=_mesh(),
      compiler_params=pltpu.CompilerParams(use_tc_tiling_on_sc=False),
      scratch_types=[
          pltpu.VMEM((NCHUNK, CHUNK), jnp.int32),
          pltpu.VMEM((NCHUNK, CHUNK), jnp.int32),
          [pltpu.VMEM((CHUNK, FH), jnp.float32)] * KDEPTH,
          pltpu.VMEM((ZR, FH), jnp.float32),
          pltpu.VMEM_SHARED((N, FH), jnp.float32),
          pltpu.SemaphoreType.DMA,
          pltpu.SemaphoreType.DMA,
      ],
  )
  def agg_kernel(ulo_hbm, uhi_hbm, src_hbm, dst_hbm, olo_hbm, ohi_hbm,
                 src_v, dst_v, rows, zero_v, acc_sh, sem_g, sem_s):
    c = lax.axis_index("c")
    s = lax.axis_index("s")
    wid = s * NC + c
    _zero_vmem(zero_v, ZR, FH)
    pltpu.sync_copy(src_hbm.at[wid], src_v)
    pltpu.sync_copy(dst_hbm.at[wid], dst_v)

    def run_group(u_hbm, base, nk):
      gd = [
          pltpu.async_copy(u_hbm.at[src_v.at[base + k]], rows[k], sem_g)
          for k in range(nk)
      ]
      for d in gd:
        d.wait()
      sd = [
          pltpu.async_copy(
              rows[k], acc_sh.at[dst_v.at[base + k]], sem_s, add=True)
          for k in range(nk)
      ]
      for d in sd:
        d.wait()

    for u_hbm, out_hbm in ((ulo_hbm, olo_hbm), (uhi_hbm, ohi_hbm)):
      _zero_shared(zero_v, acc_sh, s)
      plsc.subcore_barrier()

      def body(g, _, u_hbm=u_hbm):
        run_group(u_hbm, g * KDEPTH, KDEPTH)
        return 0

      lax.fori_loop(0, NCHUNK // KDEPTH, body, 0)
      run_group(u_hbm, (NCHUNK // KDEPTH) * KDEPTH, PEEL)
      plsc.subcore_barrier()
      _drain_shared(acc_sh, out_hbm, c, s)
      plsc.subcore_barrier()

  return agg_kernel(u_lo, u_hi, src3, dst3)


def _dinv_block(dp0, dp1):
  deg = dp0[:, 0:1] + dp1[:, 0:1] + 1.0
  return lax.rsqrt(deg)


def _tc_matmul_scale_body(x_ref, w_ref, dp0_ref, dp1_ref, ulo_ref, uhi_ref):
  dinv = _dinv_block(dp0_ref[...], dp1_ref[...])
  xw = jnp.dot(x_ref[...], w_ref[...], preferred_element_type=jnp.float32)
  u = xw * dinv
  ulo_ref[...] = u[:, :FH]
  uhi_ref[...] = u[:, FH:]


def _tc_mid_body(alo0_ref, alo1_ref, ahi0_ref, ahi1_ref, ulo_ref, uhi_ref,
                 dp0_ref, dp1_ref, b_ref, vlo_ref, vhi_ref):
  dinv = _dinv_block(dp0_ref[...], dp1_ref[...])
  t_lo = dinv * (alo0_ref[...] + alo1_ref[...] + ulo_ref[...]) + b_ref[:, :FH]
  t_hi = dinv * (ahi0_ref[...] + ahi1_ref[...] + uhi_ref[...]) + b_ref[:, FH:]
  vlo_ref[...] = dinv * jnp.maximum(t_lo, 0.0)
  vhi_ref[...] = dinv * jnp.maximum(t_hi, 0.0)


def _tc_final_body(alo0_ref, alo1_ref, ahi0_ref, ahi1_ref, vlo_ref, vhi_ref,
                   dp0_ref, dp1_ref, w_ref, b_ref, o_ref):
  dinv = _dinv_block(dp0_ref[...], dp1_ref[...])
  t_lo = dinv * (alo0_ref[...] + alo1_ref[...] + vlo_ref[...])
  t_hi = dinv * (ahi0_ref[...] + ahi1_ref[...] + vhi_ref[...])
  t = jnp.concatenate([t_lo, t_hi], axis=1)
  o_ref[...] = (
      jnp.dot(t, w_ref[...], preferred_element_type=jnp.float32) + b_ref[...])


def _row_spec(width):
  return pl.BlockSpec((ROWS_BLK, width), lambda i: (i, 0))


def _row_spec_hi(width):
  return pl.BlockSpec((ROWS_BLK, width), lambda i: (i + TC_GRID, 0))


def _full_spec(shape):
  return pl.BlockSpec(shape, lambda i: tuple(0 for _ in shape))


def _tc_matmul_scale(x, W, degp):
  return pl.pallas_call(
      _tc_matmul_scale_body,
      grid=(TC_GRID,),
      in_specs=[
          _row_spec(F),
          _full_spec((F, F)),
          _row_spec(DEGW),
          _row_spec_hi(DEGW),
      ],
      out_specs=(_row_spec(FH), _row_spec(FH)),
      out_shape=(
          jax.ShapeDtypeStruct((N, FH), jnp.float32),
          jax.ShapeDtypeStruct((N, FH), jnp.float32),
      ),
  )(x, W, degp, degp)


def _tc_mid(a_lo, a_hi, u_lo, u_hi, degp, b1):
  return pl.pallas_call(
      _tc_mid_body,
      grid=(TC_GRID,),
      in_specs=[
          _row_spec(FH),
          _row_spec_hi(FH),
          _row_spec(FH),
          _row_spec_hi(FH),
          _row_spec(FH),
          _row_spec(FH),
          _row_spec(DEGW),
          _row_spec_hi(DEGW),
          _full_spec((1, F)),
      ],
      out_specs=(_row_spec(FH), _row_spec(FH)),
      out_shape=(
          jax.ShapeDtypeStruct((N, FH), jnp.float32),
          jax.ShapeDtypeStruct((N, FH), jnp.float32),
      ),
  )(a_lo, a_lo, a_hi, a_hi, u_lo, u_hi, degp, degp, b1)


def _tc_final(a_lo, a_hi, v_lo, v_hi, degp, W, b2):
  return pl.pallas_call(
      _tc_final_body,
      grid=(TC_GRID,),
      in_specs=[
          _row_spec(FH),
          _row_spec_hi(FH),
          _row_spec(FH),
          _row_spec_hi(FH),
          _row_spec(FH),
          _row_spec(FH),
          _row_spec(DEGW),
          _row_spec_hi(DEGW),
          _full_spec((F, F)),
          _full_spec((1, F)),
      ],
      out_specs=_row_spec(F),
      out_shape=jax.ShapeDtypeStruct((N, F), jnp.float32),
  )(a_lo, a_lo, a_hi, a_hi, v_lo, v_hi, degp, degp, W, b2)


@jax.jit
def kernel(x, edge_index, W1, b1, W2, b2):
  src3 = edge_index[0].astype(jnp.int32).reshape(NW, NCHUNK, CHUNK)
  dst3 = edge_index[1].astype(jnp.int32).reshape(NW, NCHUNK, CHUNK)
  b1r = b1.reshape(1, F)
  b2r = b2.reshape(1, F)

  degp = _sc_degree(dst3)
  u_lo, u_hi = _tc_matmul_scale(x, W1, degp)
  a_lo, a_hi = _sc_agg(u_lo, u_hi, src3, dst3)
  v_lo, v_hi = _tc_mid(a_lo, a_hi, u_lo, u_hi, degp, b1r)
  b_lo, b_hi = _sc_agg(v_lo, v_hi, src3, dst3)
  return _tc_final(b_lo, b_hi, v_lo, v_hi, degp, W2, b2r)
